# Initial kernel scaffold; baseline (speedup 1.0000x reference)
import jax
import jax.numpy as jnp
from jax.experimental import pallas as pl


def kernel(adj_data, adj_indices, start_node_id):
    raise NotImplementedError("write your pallas kernel here")


# speedup vs baseline: 101.6238x; 101.6238x over previous
import functools

import jax
import jax.numpy as jnp
from jax import lax
from jax.experimental import pallas as pl
from jax.experimental.pallas import tpu as pltpu
from jax.experimental.pallas import tpu_sc as plsc

N = 10000
NP = 10240
E = 640000
K = 256
ALPHA = 0.15
RHO = 0.0001
NUM_STEPS = 16
THR = ALPHA * RHO

NC = 2
NS = 16
L = 16
SLICE = NP // NS
EPT = E // NS
CH = 8000
NCH = EPT // CH


def _hist_rank_select(hist, rank):
    def body(jj, carry):
        cum, nb, ca = carry
        j = 15 - jj
        v = hist[pl.ds(j * L, L)]
        rv = lax.rev(v, (0,))
        cs = plsc.cumsum(rv) + cum
        lt = cs < rank
        nb = nb + jnp.sum(jnp.where(lt, 1, 0))
        ca = ca + jnp.sum(jnp.where(lt, rv, 0))
        cum = jnp.max(cs)
        return cum, nb, ca

    zero = jnp.int32(0)
    _, nb, ca = lax.fori_loop(0, 16, body, (zero, zero, zero))
    bucket = 255 - nb
    return bucket, rank - ca


def _sc_body(data_h, src_h, dst_h, s_h, qout_h, accs_h, xscr_h,
             qtab, acc, ebuf_d, ebuf_s, ebuf_t, red, xful, mful, ssl, xw,
             hist):
    c = lax.axis_index("c")
    w = lax.axis_index("s")
    zeros16f = jnp.zeros((L,), jnp.float32)
    zeros16i = jnp.zeros((L,), jnp.int32)
    ones16i = jnp.ones((L,), jnp.int32)

    pltpu.sync_copy(s_h, qtab)
    pltpu.sync_copy(s_h.at[pl.ds(w * SLICE, SLICE)], ssl)

    def step(_, carry):
        def z_body(i, _):
            acc[pl.ds(i * L, L)] = zeros16f
            return 0
        lax.fori_loop(0, NP // L, z_body, 0)

        def ch_body(ch, _):
            base = w * EPT + ch * CH
            pltpu.sync_copy(data_h.at[pl.ds(base, CH)], ebuf_d)
            pltpu.sync_copy(src_h.at[pl.ds(base, CH)], ebuf_s)
            pltpu.sync_copy(dst_h.at[pl.ds(base, CH)], ebuf_t)

            def e_body(i, _):
                sl = pl.ds(i * L, L)
                sv = ebuf_s[sl]
                dv = ebuf_t[sl]
                av = ebuf_d[sl]
                qv = plsc.load_gather(qtab, [sv])
                plsc.addupdate_scatter(acc, [dv], av * qv)
                return 0
            lax.fori_loop(0, CH // L, e_body, 0)
            return 0
        lax.fori_loop(0, NCH, ch_body, 0)

        def pub_body(j, _):
            pltpu.sync_copy(acc.at[pl.ds(j * SLICE, SLICE)], accs_h.at[c, j, w])
            return 0
        lax.fori_loop(0, NS, pub_body, 0)
        plsc.subcore_barrier()

        pltpu.sync_copy(accs_h.at[c, w], red)

        def r_body(k, _):
            sl = pl.ds(k * L, L)
            tot = red[0, sl]
            for j in range(1, NS):
                tot = tot + red[j, sl]
            xw[sl] = (1.0 - ALPHA) * tot - ALPHA * ssl[sl]
            return 0
        lax.fori_loop(0, SLICE // L, r_body, 0)
        pltpu.sync_copy(xw, xscr_h.at[c, w])
        plsc.subcore_barrier()

        pltpu.sync_copy(xscr_h.at[c], xful)

        def hz_body(i, _):
            hist[pl.ds(i * L, L)] = zeros16i
            return 0
        lax.fori_loop(0, 256 // L, hz_body, 0)

        def m_body(j, _):
            def mk_body(k, _):
                sl = pl.ds(k * L, L)
                m = jnp.bitwise_and(
                    lax.bitcast_convert_type(xful[j, sl], jnp.int32),
                    jnp.int32(0x7FFFFFFF))
                mful[j, sl] = m
                b = lax.shift_right_logical(m, 23)
                plsc.addupdate_scatter(hist, [b], ones16i)
                return 0
            lax.fori_loop(0, SLICE // L, mk_body, 0)
            return 0
        lax.fori_loop(0, NS, m_body, 0)

        rank = jnp.int32(K)
        b1, rank = _hist_rank_select(hist, rank)

        def radix_pass(prefix, pref_shift, buck_shift, buck_mask, rank):
            def hz(i, _):
                hist[pl.ds(i * L, L)] = zeros16i
                return 0
            lax.fori_loop(0, 256 // L, hz, 0)

            def pj(j, _):
                def pk(k, _):
                    sl = pl.ds(k * L, L)
                    m = mful[j, sl]
                    keep = lax.shift_right_logical(m, pref_shift) == prefix
                    b = jnp.bitwise_and(
                        lax.shift_right_logical(m, buck_shift),
                        jnp.int32(buck_mask))
                    plsc.addupdate_scatter(hist, [b], ones16i, mask=keep)
                    return 0
                lax.fori_loop(0, SLICE // L, pk, 0)
                return 0
            lax.fori_loop(0, NS, pj, 0)
            return _hist_rank_select(hist, rank)

        b2, rank = radix_pass(b1, 23, 15, 0xFF, rank)
        p2 = b1 * 256 + b2
        b3, rank = radix_pass(p2, 15, 7, 0xFF, rank)
        p3 = p2 * 256 + b3
        b4, _ = radix_pass(p3, 7, 0, 0x7F, rank)
        t = p3 * 128 + b4

        def a_body(j, _):
            def ak_body(k, _):
                sl = pl.ds(k * L, L)
                x = xful[j, sl]
                m = mful[j, sl]
                val = jnp.sign(x) * jnp.maximum(jnp.abs(x) - THR, 0.0)
                qtab[pl.ds(j * SLICE + k * L, L)] = jnp.where(m >= t, val, 0.0)
                return 0
            lax.fori_loop(0, SLICE // L, ak_body, 0)
            return 0
        lax.fori_loop(0, NS, a_body, 0)
        return carry

    lax.fori_loop(0, NUM_STEPS, step, 0)

    @pl.when(c == 0)
    def _():
        pltpu.sync_copy(qtab.at[pl.ds(w * SLICE, SLICE)],
                        qout_h.at[pl.ds(w * SLICE, SLICE)])


@jax.jit
def _run(data, src, dst, svec):
    mesh = plsc.VectorSubcoreMesh(core_axis_name="c", subcore_axis_name="s",
                                  num_cores=NC, num_subcores=NS)
    f = pl.kernel(
        _sc_body,
        out_type=[
            jax.ShapeDtypeStruct((NP,), jnp.float32),
            jax.ShapeDtypeStruct((NC, NS, NS, SLICE), jnp.float32),
            jax.ShapeDtypeStruct((NC, NS, SLICE), jnp.float32),
        ],
        mesh=mesh,
        compiler_params=pltpu.CompilerParams(needs_layout_passes=False),
        scratch_types=[
            pltpu.VMEM((NP,), jnp.float32),
            pltpu.VMEM((NP,), jnp.float32),
            pltpu.VMEM((CH,), jnp.float32),
            pltpu.VMEM((CH,), jnp.int32),
            pltpu.VMEM((CH,), jnp.int32),
            pltpu.VMEM((NS, SLICE), jnp.float32),
            pltpu.VMEM((NS, SLICE), jnp.float32),
            pltpu.VMEM((NS, SLICE), jnp.int32),
            pltpu.VMEM((SLICE,), jnp.float32),
            pltpu.VMEM((SLICE,), jnp.float32),
            pltpu.VMEM((256,), jnp.int32),
        ],
    )
    q, _, _ = f(data, src, dst, svec)
    return q[:N]


def kernel(adj_data, adj_indices, start_node_id):
    src = adj_indices[:, 0].astype(jnp.int32)
    dst = adj_indices[:, 1].astype(jnp.int32)
    svec = jnp.zeros((NP,), jnp.float32).at[start_node_id].set(1.0)
    return _run(adj_data.astype(jnp.float32), src, dst, svec)


# speedup vs baseline: 279.0910x; 2.7463x over previous
import jax
import jax.numpy as jnp
from jax import lax
from jax.experimental import pallas as pl
from jax.experimental.pallas import tpu as pltpu
from jax.experimental.pallas import tpu_sc as plsc

N = 10000
NP = 10240
E = 640000
K = 256
ALPHA = 0.15
RHO = 0.0001
NUM_STEPS = 16
THR = ALPHA * RHO

NC = 2
NS = 16
L = 16
SLICE = NP // NS
EPT = E // NS
NV = SLICE // L


def _hist_rank_select(hist, rank):
    def body(jj, carry):
        cum, nb, ca = carry
        j = 15 - jj
        v = hist[pl.ds(j * L, L)]
        rv = lax.rev(v, (0,))
        cs = plsc.cumsum(rv) + cum
        lt = cs < rank
        nb = nb + jnp.sum(jnp.where(lt, 1, 0))
        ca = ca + jnp.sum(jnp.where(lt, rv, 0))
        cum = jnp.max(cs)
        return cum, nb, ca

    zero = jnp.int32(0)
    _, nb, ca = lax.fori_loop(0, 16, body, (zero, zero, zero))
    return 255 - nb, rank - ca


def _sc_body(pack_h, data_h, s_h, zeros_h, qout_h, accs_h, xscr_h,
             pbuf, abuf, qtab, acc, xful, cbuf, ssl, xw, hist, zsem):
    c = lax.axis_index("c")
    w = lax.axis_index("s")
    zeros16i = jnp.zeros((L,), jnp.int32)
    ones16i = jnp.ones((L,), jnp.int32)
    lanes = lax.broadcasted_iota(jnp.int32, (L,), 0)

    pltpu.sync_copy(pack_h.at[pl.ds(w * EPT, EPT)], pbuf)
    pltpu.sync_copy(data_h.at[pl.ds(w * EPT, EPT)], abuf)
    pltpu.sync_copy(s_h, qtab)
    pltpu.sync_copy(s_h.at[pl.ds(w * SLICE, SLICE)], ssl)
    pltpu.async_copy(zeros_h, acc, zsem)

    def _mbits(x):
        return jnp.bitwise_and(lax.bitcast_convert_type(x, jnp.int32),
                               jnp.int32(0x7FFFFFFF))

    def step(_, carry):
        pltpu.make_async_copy(zeros_h, acc, zsem).wait()

        @plsc.parallel_loop(0, EPT // L, 1, unroll=10)
        def _(i):
            sl = pl.ds(i * L, L)
            pv = pbuf[sl]
            av = abuf[sl]
            sv = jnp.bitwise_and(pv, jnp.int32(0x3FFF))
            dv = lax.shift_right_logical(pv, 14)
            qv = plsc.load_gather(qtab, [sv])
            plsc.addupdate_scatter(acc, [dv], av * qv)

        descs = [pltpu.async_copy(acc.at[pl.ds(j * SLICE, SLICE)],
                                  accs_h.at[c, j, w], zsem)
                 for j in range(NS)]
        for d in descs:
            d.wait()
        plsc.subcore_barrier()

        pltpu.sync_copy(accs_h.at[c, w], xful)
        pltpu.async_copy(zeros_h, acc, zsem)

        @plsc.parallel_loop(0, NV, 1, unroll=2)
        def _(k):
            sl = pl.ds(k * L, L)
            tot = xful[0, sl]
            for j in range(1, NS):
                tot = tot + xful[j, sl]
            xw[sl] = (1.0 - ALPHA) * tot - ALPHA * ssl[sl]

        pltpu.sync_copy(xw, xscr_h.at[c, w])
        plsc.subcore_barrier()
        pltpu.sync_copy(xscr_h.at[c], xful)

        def hz(i, _):
            hist[pl.ds(i * L, L)] = zeros16i
            return 0
        lax.fori_loop(0, 256 // L, hz, 0)

        @plsc.parallel_loop(0, NS * NV, 1, unroll=4)
        def _(i):
            m = _mbits(xful[i // NV, pl.ds((i % NV) * L, L)])
            plsc.addupdate_scatter(hist, [lax.shift_right_logical(m, 23)],
                                   ones16i)

        b1, r1 = _hist_rank_select(hist, jnp.int32(K))

        def radix_rest():
            def cj(j, off):
                def ck(k, off):
                    m = _mbits(xful[j, pl.ds(k * L, L)])
                    keep = lax.shift_right_logical(m, 23) == b1
                    plsc.store_compressed(cbuf.at[pl.ds(off, L)], m, mask=keep)
                    return off + jnp.max(plsc.all_reduce_population_count(keep))
                return lax.fori_loop(0, NV, ck, off)
            cnt = lax.fori_loop(0, NS, cj, jnp.int32(0))
            nv = (cnt + (L - 1)) // L

            def rpass(pref, pref_shift, buck_shift, buck_mask, rank):
                def hz2(i, _):
                    hist[pl.ds(i * L, L)] = zeros16i
                    return 0
                lax.fori_loop(0, 256 // L, hz2, 0)

                def pb(i, _):
                    m = cbuf[pl.ds(i * L, L)]
                    keep = jnp.logical_and(
                        lanes < cnt - i * L,
                        lax.shift_right_logical(m, pref_shift) == pref)
                    b = jnp.bitwise_and(
                        lax.shift_right_logical(m, buck_shift),
                        jnp.int32(buck_mask))
                    plsc.addupdate_scatter(hist, [b], ones16i, mask=keep)
                    return 0
                lax.fori_loop(0, nv, pb, 0)
                return _hist_rank_select(hist, rank)

            b2, r2 = rpass(b1, 23, 15, 0xFF, r1)
            p2 = b1 * 256 + b2
            b3, r3 = rpass(p2, 15, 7, 0xFF, r2)
            p3 = p2 * 256 + b3
            b4, _ = rpass(p3, 7, 0, 0x7F, r3)
            return p3 * 128 + b4

        t = lax.cond(b1 > 0, radix_rest, lambda: jnp.int32(1))

        @plsc.parallel_loop(0, NS * NV, 1, unroll=4)
        def _(i):
            x = xful[i // NV, pl.ds((i % NV) * L, L)]
            val = jnp.sign(x) * jnp.maximum(jnp.abs(x) - THR, 0.0)
            qtab[pl.ds(i * L, L)] = jnp.where(_mbits(x) >= t, val, 0.0)

        return carry

    lax.fori_loop(0, NUM_STEPS, step, 0)
    pltpu.make_async_copy(zeros_h, acc, zsem).wait()

    @pl.when(c == 0)
    def _():
        pltpu.sync_copy(qtab.at[pl.ds(w * SLICE, SLICE)],
                        qout_h.at[pl.ds(w * SLICE, SLICE)])


@jax.jit
def _run(packed, data, svec, zvec):
    mesh = plsc.VectorSubcoreMesh(core_axis_name="c", subcore_axis_name="s",
                                  num_cores=NC, num_subcores=NS)
    f = pl.kernel(
        _sc_body,
        out_type=[
            jax.ShapeDtypeStruct((NP,), jnp.float32),
            jax.ShapeDtypeStruct((NC, NS, NS, SLICE), jnp.float32),
            jax.ShapeDtypeStruct((NC, NS, SLICE), jnp.float32),
        ],
        mesh=mesh,
        compiler_params=pltpu.CompilerParams(needs_layout_passes=False),
        scratch_types=[
            pltpu.VMEM((EPT,), jnp.int32),
            pltpu.VMEM((EPT,), jnp.float32),
            pltpu.VMEM((NP,), jnp.float32),
            pltpu.VMEM((NP,), jnp.float32),
            pltpu.VMEM((NS, SLICE), jnp.float32),
            pltpu.VMEM((NP + L,), jnp.int32),
            pltpu.VMEM((SLICE,), jnp.float32),
            pltpu.VMEM((SLICE,), jnp.float32),
            pltpu.VMEM((256,), jnp.int32),
            pltpu.SemaphoreType.DMA,
        ],
    )
    q, _, _ = f(packed, data, svec, zvec)
    return q[:N]


def kernel(adj_data, adj_indices, start_node_id):
    src = adj_indices[:, 0].astype(jnp.int32)
    dst = adj_indices[:, 1].astype(jnp.int32)
    packed = jnp.bitwise_or(src, jnp.left_shift(dst, 14))
    svec = jnp.zeros((NP,), jnp.float32).at[start_node_id].set(1.0)
    zvec = jnp.zeros((NP,), jnp.float32)
    return _run(packed, adj_data.astype(jnp.float32), svec, zvec)


# speedup vs baseline: 279.7550x; 1.0024x over previous
import jax
import jax.numpy as jnp
from jax import lax
from jax.experimental import pallas as pl
from jax.experimental.pallas import tpu as pltpu
from jax.experimental.pallas import tpu_sc as plsc

N = 10000
NP = 10240
E = 640000
K = 256
ALPHA = 0.15
RHO = 0.0001
NUM_STEPS = 16
THR = ALPHA * RHO

NC = 2
NS = 16
L = 16
SLICE = NP // NS
EPT = E // NS
NV = SLICE // L


def _hist_rank_select(hist, rank):
    def body(jj, carry):
        cum, nb, ca = carry
        j = 15 - jj
        v = hist[pl.ds(j * L, L)]
        rv = lax.rev(v, (0,))
        cs = plsc.cumsum(rv) + cum
        lt = cs < rank
        nb = nb + jnp.sum(jnp.where(lt, 1, 0))
        ca = ca + jnp.sum(jnp.where(lt, rv, 0))
        cum = jnp.max(cs)
        return cum, nb, ca

    zero = jnp.int32(0)
    _, nb, ca = lax.fori_loop(0, 16, body, (zero, zero, zero))
    return 255 - nb, rank - ca


def _sc_body(pack_h, data_h, s_h, zeros_h, qout_h, accs_h, xscr_h,
             pbuf, abuf, qtab, acc, xful, cbuf, ssl, xw, hist, zsem):
    c = lax.axis_index("c")
    w = lax.axis_index("s")
    zeros16i = jnp.zeros((L,), jnp.int32)
    ones16i = jnp.ones((L,), jnp.int32)
    lanes = lax.broadcasted_iota(jnp.int32, (L,), 0)

    pltpu.sync_copy(pack_h.at[pl.ds(w * EPT, EPT)], pbuf)
    pltpu.sync_copy(data_h.at[pl.ds(w * EPT, EPT)], abuf)
    pltpu.sync_copy(s_h, qtab)
    pltpu.sync_copy(s_h.at[pl.ds(w * SLICE, SLICE)], ssl)
    pltpu.async_copy(zeros_h, acc, zsem)

    def _mbits(x):
        return jnp.bitwise_and(lax.bitcast_convert_type(x, jnp.int32),
                               jnp.int32(0x7FFFFFFF))

    def step(_, carry):
        pltpu.make_async_copy(zeros_h, acc, zsem).wait()

        with jax.named_scope("sweep"):
            @plsc.parallel_loop(0, EPT // L, 1, unroll=10)
            def _(i):
                sl = pl.ds(i * L, L)
                pv = pbuf[sl]
                av = abuf[sl]
                sv = jnp.bitwise_and(pv, jnp.int32(0x3FFF))
                dv = lax.shift_right_logical(pv, 14)
                qv = plsc.load_gather(qtab, [sv])
                plsc.addupdate_scatter(acc, [dv], av * qv)

        with jax.named_scope("publish"):
            descs = [pltpu.async_copy(acc.at[pl.ds(j * SLICE, SLICE)],
                                      accs_h.at[c, j, w], zsem)
                     for j in range(NS)]
            for d in descs:
                d.wait()
            plsc.subcore_barrier()

        with jax.named_scope("reduce"):
            pltpu.sync_copy(accs_h.at[c, w], xful)
            pltpu.async_copy(zeros_h, acc, zsem)

            @plsc.parallel_loop(0, NV, 1, unroll=2)
            def _(k):
                sl = pl.ds(k * L, L)
                tot = xful[0, sl]
                for j in range(1, NS):
                    tot = tot + xful[j, sl]
                xw[sl] = (1.0 - ALPHA) * tot - ALPHA * ssl[sl]

            pltpu.sync_copy(xw, xscr_h.at[c, w])
            plsc.subcore_barrier()

        with jax.named_scope("bcast"):
            pltpu.sync_copy(xscr_h.at[c], xful)

        with jax.named_scope("hist1"):
            def hz(i, _):
                hist[pl.ds(i * L, L)] = zeros16i
                return 0
            lax.fori_loop(0, 256 // L, hz, 0)

            @plsc.parallel_loop(0, NS * NV, 1, unroll=4)
            def _(i):
                m = _mbits(xful[i // NV, pl.ds((i % NV) * L, L)])
                plsc.addupdate_scatter(hist, [lax.shift_right_logical(m, 23)],
                                       ones16i)

            b1, r1 = _hist_rank_select(hist, jnp.int32(K))

        def radix_rest():
            def cj(j, off):
                def ck(k, off):
                    m = _mbits(xful[j, pl.ds(k * L, L)])
                    keep = lax.shift_right_logical(m, 23) == b1
                    plsc.store_compressed(cbuf.at[pl.ds(off, L)], m, mask=keep)
                    return off + jnp.max(plsc.all_reduce_population_count(keep))
                return lax.fori_loop(0, NV, ck, off)
            cnt = lax.fori_loop(0, NS, cj, jnp.int32(0))
            nv = (cnt + (L - 1)) // L

            def rpass(pref, pref_shift, buck_shift, buck_mask, rank):
                def hz2(i, _):
                    hist[pl.ds(i * L, L)] = zeros16i
                    return 0
                lax.fori_loop(0, 256 // L, hz2, 0)

                def pb(i, _):
                    m = cbuf[pl.ds(i * L, L)]
                    keep = jnp.logical_and(
                        lanes < cnt - i * L,
                        lax.shift_right_logical(m, pref_shift) == pref)
                    b = jnp.bitwise_and(
                        lax.shift_right_logical(m, buck_shift),
                        jnp.int32(buck_mask))
                    plsc.addupdate_scatter(hist, [b], ones16i, mask=keep)
                    return 0
                lax.fori_loop(0, nv, pb, 0)
                return _hist_rank_select(hist, rank)

            b2, r2 = rpass(b1, 23, 15, 0xFF, r1)
            p2 = b1 * 256 + b2
            b3, r3 = rpass(p2, 15, 7, 0xFF, r2)
            p3 = p2 * 256 + b3
            b4, _ = rpass(p3, 7, 0, 0x7F, r3)
            return p3 * 128 + b4

        with jax.named_scope("radix"):
            t = lax.cond(b1 > 0, radix_rest, lambda: jnp.int32(1))

        with jax.named_scope("apply"):
            @plsc.parallel_loop(0, NS * NV, 1, unroll=4)
            def _(i):
                x = xful[i // NV, pl.ds((i % NV) * L, L)]
                val = jnp.sign(x) * jnp.maximum(jnp.abs(x) - THR, 0.0)
                qtab[pl.ds(i * L, L)] = jnp.where(_mbits(x) >= t, val, 0.0)

        return carry

    lax.fori_loop(0, NUM_STEPS, step, 0)
    pltpu.make_async_copy(zeros_h, acc, zsem).wait()

    @pl.when(c == 0)
    def _():
        pltpu.sync_copy(qtab.at[pl.ds(w * SLICE, SLICE)],
                        qout_h.at[pl.ds(w * SLICE, SLICE)])


@jax.jit
def _run(packed, data, svec, zvec):
    mesh = plsc.VectorSubcoreMesh(core_axis_name="c", subcore_axis_name="s",
                                  num_cores=NC, num_subcores=NS)
    f = pl.kernel(
        _sc_body,
        out_type=[
            jax.ShapeDtypeStruct((NP,), jnp.float32),
            jax.ShapeDtypeStruct((NC, NS, NS, SLICE), jnp.float32),
            jax.ShapeDtypeStruct((NC, NS, SLICE), jnp.float32),
        ],
        mesh=mesh,
        compiler_params=pltpu.CompilerParams(needs_layout_passes=False),
        scratch_types=[
            pltpu.VMEM((EPT,), jnp.int32),
            pltpu.VMEM((EPT,), jnp.float32),
            pltpu.VMEM((NP,), jnp.float32),
            pltpu.VMEM((NP,), jnp.float32),
            pltpu.VMEM((NS, SLICE), jnp.float32),
            pltpu.VMEM((NP + L,), jnp.int32),
            pltpu.VMEM((SLICE,), jnp.float32),
            pltpu.VMEM((SLICE,), jnp.float32),
            pltpu.VMEM((256,), jnp.int32),
            pltpu.SemaphoreType.DMA,
        ],
    )
    q, _, _ = f(packed, data, svec, zvec)
    return q[:N]


def kernel(adj_data, adj_indices, start_node_id):
    src = adj_indices[:, 0].astype(jnp.int32)
    dst = adj_indices[:, 1].astype(jnp.int32)
    packed = jnp.bitwise_or(src, jnp.left_shift(dst, 14))
    svec = jnp.zeros((NP,), jnp.float32).at[start_node_id].set(1.0)
    zvec = jnp.zeros((NP,), jnp.float32)
    return _run(packed, adj_data.astype(jnp.float32), svec, zvec)


# speedup vs baseline: 353.4999x; 1.2636x over previous
import jax
import jax.numpy as jnp
from jax import lax
from jax.experimental import pallas as pl
from jax.experimental.pallas import tpu as pltpu
from jax.experimental.pallas import tpu_sc as plsc

N = 10000
NP = 10240
E = 640000
K = 256
ALPHA = 0.15
RHO = 0.0001
NUM_STEPS = 16
THR = ALPHA * RHO

NC = 2
NS = 16
L = 16
SLICE = NP // NS
EPT = E // NS
NV = SLICE // L


def _hist_rank_select(hist, rank):
    def body(j, carry):
        cum, nb, ca = carry
        v = hist[pl.ds(j * L, L)]
        cs = plsc.cumsum(v) + cum
        lt = cs < rank
        nb = nb + plsc.all_reduce_population_count(lt)
        ca = ca + jnp.sum(jnp.where(lt, v, 0))
        return jnp.max(cs), nb, ca

    zero = jnp.int32(0)
    _, nb, ca = lax.fori_loop(0, 16, body,
                              (zero, jnp.zeros((L,), jnp.int32), zero))
    return 255 - nb, rank - ca


def _sc_body(pack_h, data_h, s_h, zeros_h, qout_h, accs_h, xscr_h, hists_h,
             pbuf, abuf, qtab, acc, xful, cbuf, hbuf, ssl, xw, hist, zsem):
    c = lax.axis_index("c")
    w = lax.axis_index("s")
    zeros16i = jnp.zeros((L,), jnp.int32)
    ones16i = jnp.ones((L,), jnp.int32)
    lanes = lax.broadcasted_iota(jnp.int32, (L,), 0)

    pltpu.sync_copy(pack_h.at[pl.ds(w * EPT, EPT)], pbuf)
    pltpu.sync_copy(data_h.at[pl.ds(w * EPT, EPT)], abuf)
    pltpu.sync_copy(s_h, qtab)
    pltpu.sync_copy(s_h.at[pl.ds(w * SLICE, SLICE)], ssl)
    pltpu.async_copy(zeros_h, acc, zsem)

    def _mbits(x):
        return jnp.bitwise_and(lax.bitcast_convert_type(x, jnp.int32),
                               jnp.int32(0x7FFFFFFF))

    def step(_, carry):
        pltpu.make_async_copy(zeros_h, acc, zsem).wait()

        @plsc.parallel_loop(0, EPT // L, 1, unroll=16)
        def _(i):
            sl = pl.ds(i * L, L)
            pv = pbuf[sl]
            av = abuf[sl]
            sv = jnp.bitwise_and(pv, jnp.int32(0x3FFF))
            dv = lax.shift_right_logical(pv, 14)
            qv = plsc.load_gather(qtab, [sv])
            plsc.addupdate_scatter(acc, [dv], av * qv)

        descs = [pltpu.async_copy(acc.at[pl.ds(j * SLICE, SLICE)],
                                  accs_h.at[c, j, w], zsem)
                 for j in range(NS)]
        for d in descs:
            d.wait()
        plsc.subcore_barrier()

        pltpu.sync_copy(accs_h.at[c, w], xful)
        pltpu.async_copy(zeros_h, acc, zsem)

        def hz(i, _):
            hist[pl.ds(i * L, L)] = zeros16i
            return 0
        lax.fori_loop(0, 256 // L, hz, 0)

        def red(k, _):
            sl = pl.ds(k * L, L)
            tot = xful[0, sl]
            for j in range(1, NS):
                tot = tot + xful[j, sl]
            xv = (1.0 - ALPHA) * tot - ALPHA * ssl[sl]
            xw[sl] = xv
            b = 255 - lax.shift_right_logical(_mbits(xv), 23)
            plsc.addupdate_scatter(hist, [b], ones16i)
            return 0
        lax.fori_loop(0, NV, red, 0)

        pltpu.sync_copy(xw, xscr_h.at[c, w])
        pltpu.sync_copy(hist, hists_h.at[c, w])
        plsc.subcore_barrier()
        pltpu.sync_copy(xscr_h.at[c], xful)
        pltpu.sync_copy(hists_h.at[c], hbuf)

        def hs(i, _):
            sl = pl.ds(i * L, L)
            tot = hbuf[0, sl]
            for j in range(1, NS):
                tot = tot + hbuf[j, sl]
            hist[sl] = tot
            return 0
        lax.fori_loop(0, 256 // L, hs, 0)

        b1, r1 = _hist_rank_select(hist, jnp.full((L,), K, jnp.int32))

        def radix_rest():
            @plsc.parallel_loop(0, NS * NV, 1, unroll=4,
                                carry=jnp.zeros((L,), jnp.int32))
            def cnt_v(i, off):
                m = _mbits(xful[i // NV, pl.ds((i % NV) * L, L)])
                keep = lax.shift_right_logical(m, 23) == b1
                ki = keep.astype(jnp.int32)
                cs = plsc.cumsum(ki)
                plsc.store_scatter(cbuf, [off + cs - ki], m, mask=keep)
                return off + plsc.all_reduce_population_count(keep)

            nv = jnp.max((cnt_v + (L - 1)) // L)

            def rpass(pref, pref_shift, buck_shift, buck_mask, rank):
                def hz2(i, _):
                    hist[pl.ds(i * L, L)] = zeros16i
                    return 0
                lax.fori_loop(0, 256 // L, hz2, 0)

                def pb(i, _):
                    m = cbuf[pl.ds(i * L, L)]
                    keep = jnp.logical_and(
                        lanes < cnt_v - i * L,
                        lax.shift_right_logical(m, pref_shift) == pref)
                    b = 255 - jnp.bitwise_and(
                        lax.shift_right_logical(m, buck_shift),
                        jnp.int32(buck_mask))
                    plsc.addupdate_scatter(hist, [b], ones16i, mask=keep)
                    return 0
                lax.fori_loop(0, nv, pb, 0)
                return _hist_rank_select(hist, rank)

            b2, r2 = rpass(b1, 23, 15, 0xFF, r1)
            p2 = b1 * 256 + b2
            b3, r3 = rpass(p2, 15, 7, 0xFF, r2)
            p3 = p2 * 256 + b3
            b4, _ = rpass(p3, 7, 0, 0x7F, r3)
            return p3 * 128 + b4

        t = lax.cond(jnp.max(b1) > 0, radix_rest,
                     lambda: jnp.ones((L,), jnp.int32))

        @plsc.parallel_loop(0, NS * NV, 1, unroll=4)
        def _(i):
            x = xful[i // NV, pl.ds((i % NV) * L, L)]
            val = jnp.sign(x) * jnp.maximum(jnp.abs(x) - THR, 0.0)
            qtab[pl.ds(i * L, L)] = jnp.where(_mbits(x) >= t, val, 0.0)

        return carry

    lax.fori_loop(0, NUM_STEPS, step, 0)
    pltpu.make_async_copy(zeros_h, acc, zsem).wait()

    @pl.when(c == 0)
    def _():
        pltpu.sync_copy(qtab.at[pl.ds(w * SLICE, SLICE)],
                        qout_h.at[pl.ds(w * SLICE, SLICE)])


@jax.jit
def _run(packed, data, svec, zvec):
    mesh = plsc.VectorSubcoreMesh(core_axis_name="c", subcore_axis_name="s",
                                  num_cores=NC, num_subcores=NS)
    f = pl.kernel(
        _sc_body,
        out_type=[
            jax.ShapeDtypeStruct((NP,), jnp.float32),
            jax.ShapeDtypeStruct((NC, NS, NS, SLICE), jnp.float32),
            jax.ShapeDtypeStruct((NC, NS, SLICE), jnp.float32),
            jax.ShapeDtypeStruct((NC, NS, 256), jnp.int32),
        ],
        mesh=mesh,
        compiler_params=pltpu.CompilerParams(needs_layout_passes=False),
        scratch_types=[
            pltpu.VMEM((EPT,), jnp.int32),
            pltpu.VMEM((EPT,), jnp.float32),
            pltpu.VMEM((NP,), jnp.float32),
            pltpu.VMEM((NP,), jnp.float32),
            pltpu.VMEM((NS, SLICE), jnp.float32),
            pltpu.VMEM((NP + L,), jnp.int32),
            pltpu.VMEM((NS, 256), jnp.int32),
            pltpu.VMEM((SLICE,), jnp.float32),
            pltpu.VMEM((SLICE,), jnp.float32),
            pltpu.VMEM((256,), jnp.int32),
            pltpu.SemaphoreType.DMA,
        ],
    )
    q, _, _, _ = f(packed, data, svec, zvec)
    return q[:N]


def kernel(adj_data, adj_indices, start_node_id):
    src = adj_indices[:, 0].astype(jnp.int32)
    dst = adj_indices[:, 1].astype(jnp.int32)
    packed = jnp.bitwise_or(src, jnp.left_shift(dst, 14))
    svec = jnp.zeros((NP,), jnp.float32).at[start_node_id].set(1.0)
    zvec = jnp.zeros((NP,), jnp.float32)
    return _run(packed, adj_data.astype(jnp.float32), svec, zvec)


# speedup vs baseline: 399.8081x; 1.1310x over previous
import jax
import jax.numpy as jnp
from jax import lax
from jax.experimental import pallas as pl
from jax.experimental.pallas import tpu as pltpu
from jax.experimental.pallas import tpu_sc as plsc

N = 10000
NP = 10240
E = 640000
K = 256
ALPHA = 0.15
RHO = 0.0001
NUM_STEPS = 16
THR = ALPHA * RHO

NC = 2
NS = 16
L = 16
SLICE = NP // NS
EPT = E // NS
NV = SLICE // L


def _hist_rank_select(hist, rank):
    def body(j, carry):
        cum, nb, ca = carry
        v = hist[pl.ds(j * L, L)]
        cs = plsc.cumsum(v) + cum
        lt = cs < rank
        nb = nb + plsc.all_reduce_population_count(lt)
        ca = ca + jnp.sum(jnp.where(lt, v, 0))
        return jnp.max(cs), nb, ca

    zero = jnp.int32(0)
    _, nb, ca = lax.fori_loop(0, 16, body,
                              (zero, jnp.zeros((L,), jnp.int32), zero))
    return 255 - nb, rank - ca


def _sc_body(pack_h, data_h, s_h, zeros_h, qout_h, accs_h,
             pbuf, abuf, qtab, acc, xful, cbuf, hbuf, ssl, xw, hist,
             xscr_h, hists_h, zsem):
    c = lax.axis_index("c")
    w = lax.axis_index("s")
    zeros16i = jnp.zeros((L,), jnp.int32)
    ones16i = jnp.ones((L,), jnp.int32)
    lanes = lax.broadcasted_iota(jnp.int32, (L,), 0)

    pltpu.sync_copy(pack_h.at[pl.ds(w * EPT, EPT)], pbuf)
    pltpu.sync_copy(data_h.at[pl.ds(w * EPT, EPT)], abuf)
    pltpu.sync_copy(s_h, qtab)
    pltpu.sync_copy(s_h.at[pl.ds(w * SLICE, SLICE)], ssl)
    pltpu.async_copy(zeros_h, acc, zsem)

    def _mbits(x):
        return jnp.bitwise_and(lax.bitcast_convert_type(x, jnp.int32),
                               jnp.int32(0x7FFFFFFF))

    def step(_, carry):
        pltpu.make_async_copy(zeros_h, acc, zsem).wait()

        @plsc.parallel_loop(0, EPT // L, 1, unroll=16)
        def _(i):
            sl = pl.ds(i * L, L)
            pv = pbuf[sl]
            av = abuf[sl]
            sv = jnp.bitwise_and(pv, jnp.int32(0x3FFF))
            dv = lax.shift_right_logical(pv, 14)
            qv = plsc.load_gather(qtab, [sv])
            plsc.addupdate_scatter(acc, [dv], av * qv)

        descs = [pltpu.async_copy(acc.at[pl.ds(j * SLICE, SLICE)],
                                  accs_h.at[c, j, w], zsem)
                 for j in range(NS)]
        for d in descs:
            d.wait()
        plsc.subcore_barrier()

        pltpu.sync_copy(accs_h.at[c, w], xful)
        pltpu.async_copy(zeros_h, acc, zsem)

        def hz(i, _):
            hist[pl.ds(i * L, L)] = zeros16i
            return 0
        lax.fori_loop(0, 256 // L, hz, 0)

        def red(k, _):
            sl = pl.ds(k * L, L)
            tot = xful[0, sl]
            for j in range(1, NS):
                tot = tot + xful[j, sl]
            xv = (1.0 - ALPHA) * tot - ALPHA * ssl[sl]
            xw[sl] = xv
            b = 255 - lax.shift_right_logical(_mbits(xv), 23)
            plsc.addupdate_scatter(hist, [b], ones16i)
            return 0
        lax.fori_loop(0, NV, red, 0)

        pltpu.sync_copy(xw, xscr_h.at[w])
        pltpu.sync_copy(hist, hists_h.at[w])
        plsc.subcore_barrier()
        pltpu.sync_copy(xscr_h, xful)
        pltpu.sync_copy(hists_h, hbuf)

        def hs(i, _):
            sl = pl.ds(i * L, L)
            tot = hbuf[0, sl]
            for j in range(1, NS):
                tot = tot + hbuf[j, sl]
            hist[sl] = tot
            return 0
        lax.fori_loop(0, 256 // L, hs, 0)

        b1, r1 = _hist_rank_select(hist, jnp.full((L,), K, jnp.int32))

        def radix_rest():
            @plsc.parallel_loop(0, NS * NV, 1, unroll=4,
                                carry=jnp.zeros((L,), jnp.int32))
            def cnt_v(i, off):
                m = _mbits(xful[i // NV, pl.ds((i % NV) * L, L)])
                keep = lax.shift_right_logical(m, 23) == b1
                ki = keep.astype(jnp.int32)
                cs = plsc.cumsum(ki)
                plsc.store_scatter(cbuf, [off + cs - ki], m, mask=keep)
                return off + plsc.all_reduce_population_count(keep)

            nv = jnp.max((cnt_v + (L - 1)) // L)

            def rpass(pref, pref_shift, buck_shift, buck_mask, rank):
                def hz2(i, _):
                    hist[pl.ds(i * L, L)] = zeros16i
                    return 0
                lax.fori_loop(0, 256 // L, hz2, 0)

                def pb(i, _):
                    m = cbuf[pl.ds(i * L, L)]
                    keep = jnp.logical_and(
                        lanes < cnt_v - i * L,
                        lax.shift_right_logical(m, pref_shift) == pref)
                    b = 255 - jnp.bitwise_and(
                        lax.shift_right_logical(m, buck_shift),
                        jnp.int32(buck_mask))
                    plsc.addupdate_scatter(hist, [b], ones16i, mask=keep)
                    return 0
                lax.fori_loop(0, nv, pb, 0)
                return _hist_rank_select(hist, rank)

            b2, r2 = rpass(b1, 23, 15, 0xFF, r1)
            p2 = b1 * 256 + b2
            b3, r3 = rpass(p2, 15, 7, 0xFF, r2)
            p3 = p2 * 256 + b3
            b4, _ = rpass(p3, 7, 0, 0x7F, r3)
            return p3 * 128 + b4

        t = lax.cond(jnp.max(b1) > 0, radix_rest,
                     lambda: jnp.ones((L,), jnp.int32))

        @plsc.parallel_loop(0, NS * NV, 1, unroll=4)
        def _(i):
            x = xful[i // NV, pl.ds((i % NV) * L, L)]
            val = jnp.sign(x) * jnp.maximum(jnp.abs(x) - THR, 0.0)
            qtab[pl.ds(i * L, L)] = jnp.where(_mbits(x) >= t, val, 0.0)

        return carry

    lax.fori_loop(0, NUM_STEPS, step, 0)
    pltpu.make_async_copy(zeros_h, acc, zsem).wait()

    @pl.when(c == 0)
    def _():
        pltpu.sync_copy(qtab.at[pl.ds(w * SLICE, SLICE)],
                        qout_h.at[pl.ds(w * SLICE, SLICE)])


@jax.jit
def _run(packed, data, svec, zvec):
    mesh = plsc.VectorSubcoreMesh(core_axis_name="c", subcore_axis_name="s",
                                  num_cores=NC, num_subcores=NS)
    f = pl.kernel(
        _sc_body,
        out_type=[
            jax.ShapeDtypeStruct((NP,), jnp.float32),
            jax.ShapeDtypeStruct((NC, NS, NS, SLICE), jnp.float32),
        ],
        mesh=mesh,
        compiler_params=pltpu.CompilerParams(needs_layout_passes=False),
        scratch_types=[
            pltpu.VMEM((EPT,), jnp.int32),
            pltpu.VMEM((EPT,), jnp.float32),
            pltpu.VMEM((NP,), jnp.float32),
            pltpu.VMEM((NP,), jnp.float32),
            pltpu.VMEM((NS, SLICE), jnp.float32),
            pltpu.VMEM((NP + L,), jnp.int32),
            pltpu.VMEM((NS, 256), jnp.int32),
            pltpu.VMEM((SLICE,), jnp.float32),
            pltpu.VMEM((SLICE,), jnp.float32),
            pltpu.VMEM((256,), jnp.int32),
            pltpu.VMEM_SHARED((NS, SLICE), jnp.float32),
            pltpu.VMEM_SHARED((NS, 256), jnp.int32),
            pltpu.SemaphoreType.DMA,
        ],
    )
    q, _ = f(packed, data, svec, zvec)
    return q[:N]


def kernel(adj_data, adj_indices, start_node_id):
    src = adj_indices[:, 0].astype(jnp.int32)
    dst = adj_indices[:, 1].astype(jnp.int32)
    packed = jnp.bitwise_or(src, jnp.left_shift(dst, 14))
    svec = jnp.zeros((NP,), jnp.float32).at[start_node_id].set(1.0)
    zvec = jnp.zeros((NP,), jnp.float32)
    return _run(packed, adj_data.astype(jnp.float32), svec, zvec)


# speedup vs baseline: 399.8901x; 1.0002x over previous
import jax
import jax.numpy as jnp
from jax import lax
from jax.experimental import pallas as pl
from jax.experimental.pallas import tpu as pltpu
from jax.experimental.pallas import tpu_sc as plsc

N = 10000
NP = 10240
E = 640000
K = 256
ALPHA = 0.15
RHO = 0.0001
NUM_STEPS = 16
THR = ALPHA * RHO

NC = 2
NS = 16
L = 16
SLICE = NP // NS
EPT = E // NS
NV = SLICE // L


def _hist_rank_select(hist, rank):
    def body(j, carry):
        cum, nb, ca = carry
        v = hist[pl.ds(j * L, L)]
        cs = plsc.cumsum(v) + cum
        lt = cs < rank
        nb = nb + plsc.all_reduce_population_count(lt)
        ca = ca + jnp.sum(jnp.where(lt, v, 0))
        return jnp.max(cs), nb, ca

    zero = jnp.int32(0)
    _, nb, ca = lax.fori_loop(0, 16, body,
                              (zero, jnp.zeros((L,), jnp.int32), zero))
    return 255 - nb, rank - ca


def _sc_body(pack_h, data_h, s_h, zeros_h, qout_h, accs_h,
             pbuf, abuf, qtab, acc, xful, cbuf, hbuf, ssl, xw, hist,
             xscr_h, hists_h, zsem):
    c = lax.axis_index("c")
    w = lax.axis_index("s")
    zeros16i = jnp.zeros((L,), jnp.int32)
    ones16i = jnp.ones((L,), jnp.int32)
    lanes = lax.broadcasted_iota(jnp.int32, (L,), 0)

    pltpu.sync_copy(pack_h.at[pl.ds(w * EPT, EPT)], pbuf)
    pltpu.sync_copy(data_h.at[pl.ds(w * EPT, EPT)], abuf)
    pltpu.sync_copy(s_h, qtab)
    pltpu.sync_copy(s_h.at[pl.ds(w * SLICE, SLICE)], ssl)
    pltpu.async_copy(zeros_h, acc, zsem)

    def _mbits(x):
        return jnp.bitwise_and(lax.bitcast_convert_type(x, jnp.int32),
                               jnp.int32(0x7FFFFFFF))

    def step(_, carry):
        pltpu.make_async_copy(zeros_h, acc, zsem).wait()

        @plsc.parallel_loop(0, EPT // L, 1, unroll=16)
        def _(i):
            sl = pl.ds(i * L, L)
            pv = pbuf[sl]
            av = abuf[sl]
            sv = jnp.bitwise_and(pv, jnp.int32(0x3FFF))
            dv = lax.shift_right_logical(pv, 14)
            qv = plsc.load_gather(qtab, [sv])
            plsc.addupdate_scatter(acc, [dv], av * qv)

        pltpu.sync_copy(acc, accs_h.at[c, w])
        plsc.subcore_barrier()

        pltpu.sync_copy(accs_h.at[c, :, pl.ds(w * SLICE, SLICE)], xful)
        pltpu.async_copy(zeros_h, acc, zsem)

        def hz(i, _):
            hist[pl.ds(i * L, L)] = zeros16i
            return 0
        lax.fori_loop(0, 256 // L, hz, 0)

        def red(k, _):
            sl = pl.ds(k * L, L)
            tot = xful[0, sl]
            for j in range(1, NS):
                tot = tot + xful[j, sl]
            xv = (1.0 - ALPHA) * tot - ALPHA * ssl[sl]
            xw[sl] = xv
            b = 255 - lax.shift_right_logical(_mbits(xv), 23)
            plsc.addupdate_scatter(hist, [b], ones16i)
            return 0
        lax.fori_loop(0, NV, red, 0)

        pltpu.sync_copy(xw, xscr_h.at[w])
        pltpu.sync_copy(hist, hists_h.at[w])
        plsc.subcore_barrier()
        pltpu.sync_copy(xscr_h, xful)
        pltpu.sync_copy(hists_h, hbuf)

        def hs(i, _):
            sl = pl.ds(i * L, L)
            tot = hbuf[0, sl]
            for j in range(1, NS):
                tot = tot + hbuf[j, sl]
            hist[sl] = tot
            return 0
        lax.fori_loop(0, 256 // L, hs, 0)

        b1, r1 = _hist_rank_select(hist, jnp.full((L,), K, jnp.int32))

        def radix_rest():
            @plsc.parallel_loop(0, NS * NV, 1, unroll=4,
                                carry=jnp.zeros((L,), jnp.int32))
            def cnt_v(i, off):
                m = _mbits(xful[i // NV, pl.ds((i % NV) * L, L)])
                keep = lax.shift_right_logical(m, 23) == b1
                ki = keep.astype(jnp.int32)
                cs = plsc.cumsum(ki)
                plsc.store_scatter(cbuf, [off + cs - ki], m, mask=keep)
                return off + plsc.all_reduce_population_count(keep)

            nv = jnp.max((cnt_v + (L - 1)) // L)

            def rpass(pref, pref_shift, buck_shift, buck_mask, rank):
                def hz2(i, _):
                    hist[pl.ds(i * L, L)] = zeros16i
                    return 0
                lax.fori_loop(0, 256 // L, hz2, 0)

                def pb(i, _):
                    m = cbuf[pl.ds(i * L, L)]
                    keep = jnp.logical_and(
                        lanes < cnt_v - i * L,
                        lax.shift_right_logical(m, pref_shift) == pref)
                    b = 255 - jnp.bitwise_and(
                        lax.shift_right_logical(m, buck_shift),
                        jnp.int32(buck_mask))
                    plsc.addupdate_scatter(hist, [b], ones16i, mask=keep)
                    return 0
                lax.fori_loop(0, nv, pb, 0)
                return _hist_rank_select(hist, rank)

            b2, r2 = rpass(b1, 23, 15, 0xFF, r1)
            p2 = b1 * 256 + b2
            b3, r3 = rpass(p2, 15, 7, 0xFF, r2)
            p3 = p2 * 256 + b3
            b4, _ = rpass(p3, 7, 0, 0x7F, r3)
            return p3 * 128 + b4

        t = lax.cond(jnp.max(b1) > 0, radix_rest,
                     lambda: jnp.ones((L,), jnp.int32))

        @plsc.parallel_loop(0, NS * NV, 1, unroll=4)
        def _(i):
            x = xful[i // NV, pl.ds((i % NV) * L, L)]
            val = jnp.sign(x) * jnp.maximum(jnp.abs(x) - THR, 0.0)
            qtab[pl.ds(i * L, L)] = jnp.where(_mbits(x) >= t, val, 0.0)

        return carry

    lax.fori_loop(0, NUM_STEPS, step, 0)
    pltpu.make_async_copy(zeros_h, acc, zsem).wait()

    @pl.when(c == 0)
    def _():
        pltpu.sync_copy(qtab.at[pl.ds(w * SLICE, SLICE)],
                        qout_h.at[pl.ds(w * SLICE, SLICE)])


@jax.jit
def _run(packed, data, svec, zvec):
    mesh = plsc.VectorSubcoreMesh(core_axis_name="c", subcore_axis_name="s",
                                  num_cores=NC, num_subcores=NS)
    f = pl.kernel(
        _sc_body,
        out_type=[
            jax.ShapeDtypeStruct((NP,), jnp.float32),
            jax.ShapeDtypeStruct((NC, NS, NP), jnp.float32),
        ],
        mesh=mesh,
        compiler_params=pltpu.CompilerParams(needs_layout_passes=False),
        scratch_types=[
            pltpu.VMEM((EPT,), jnp.int32),
            pltpu.VMEM((EPT,), jnp.float32),
            pltpu.VMEM((NP,), jnp.float32),
            pltpu.VMEM((NP,), jnp.float32),
            pltpu.VMEM((NS, SLICE), jnp.float32),
            pltpu.VMEM((NP + L,), jnp.int32),
            pltpu.VMEM((NS, 256), jnp.int32),
            pltpu.VMEM((SLICE,), jnp.float32),
            pltpu.VMEM((SLICE,), jnp.float32),
            pltpu.VMEM((256,), jnp.int32),
            pltpu.VMEM_SHARED((NS, SLICE), jnp.float32),
            pltpu.VMEM_SHARED((NS, 256), jnp.int32),
            pltpu.SemaphoreType.DMA,
        ],
    )
    q, _ = f(packed, data, svec, zvec)
    return q[:N]


def kernel(adj_data, adj_indices, start_node_id):
    src = adj_indices[:, 0].astype(jnp.int32)
    dst = adj_indices[:, 1].astype(jnp.int32)
    packed = jnp.bitwise_or(src, jnp.left_shift(dst, 14))
    svec = jnp.zeros((NP,), jnp.float32).at[start_node_id].set(1.0)
    zvec = jnp.zeros((NP,), jnp.float32)
    return _run(packed, adj_data.astype(jnp.float32), svec, zvec)


# speedup vs baseline: 422.1640x; 1.0557x over previous
import jax
import jax.numpy as jnp
from jax import lax
from jax.experimental import pallas as pl
from jax.experimental.pallas import tpu as pltpu
from jax.experimental.pallas import tpu_sc as plsc

N = 10000
NP = 10240
E = 640000
K = 256
ALPHA = 0.15
RHO = 0.0001
NUM_STEPS = 16
THR = ALPHA * RHO

NC = 2
NS = 16
L = 16
NR = NP // 128
RPT = NR // NS
NVR = NP // L
EPT = E // NS


def _hist_rank_select(hist, rank):
    def body(j, carry):
        cum, nb, ca = carry
        v = hist[pl.ds(j * L, L)]
        cs = plsc.cumsum(v) + cum
        lt = cs < rank
        nb = nb + plsc.all_reduce_population_count(lt)
        ca = ca + jnp.sum(jnp.where(lt, v, 0))
        return jnp.max(cs), nb, ca

    zero = jnp.int32(0)
    _, nb, ca = lax.fori_loop(0, 16, body,
                              (zero, jnp.zeros((L,), jnp.int32), zero))
    return 255 - nb, rank - ca


def _sc_body(pack_h, data_h, s_h, start_h, zeros_h, qout_h,
             pbuf, abuf, qtab, acc, xful, cbuf, idxb, sbuf, hist,
             xacc, zsem):
    c = lax.axis_index("c")
    w = lax.axis_index("s")
    zeros16i = jnp.zeros((L,), jnp.int32)
    ones16i = jnp.ones((L,), jnp.int32)
    lanes = lax.broadcasted_iota(jnp.int32, (L,), 0)

    pltpu.sync_copy(pack_h.at[pl.ds(w * EPT, EPT)], pbuf)
    pltpu.sync_copy(data_h.at[pl.ds(w * EPT, EPT)], abuf)
    pltpu.sync_copy(s_h, qtab)
    pltpu.sync_copy(start_h, sbuf)
    pltpu.async_copy(zeros_h, acc, zsem)

    for k in range(NR // L):
        idxb[pl.ds(k * L, L)] = k * L + lanes

    @pl.when(w < 10)
    def _():
        for par in range(2):
            pltpu.sync_copy(zeros_h.at[pl.ds(w * 8, 8), :],
                            xacc.at[par, pl.ds(w * 8, 8), :])
    plsc.subcore_barrier()

    def _mbits(x):
        return jnp.bitwise_and(lax.bitcast_convert_type(x, jnp.int32),
                               jnp.int32(0x7FFFFFFF))

    def step(si, carry):
        p = jnp.bitwise_and(si, 1)
        pltpu.make_async_copy(zeros_h, acc, zsem).wait()

        @plsc.parallel_loop(0, EPT // L, 1, unroll=16)
        def _(i):
            sl = pl.ds(i * L, L)
            pv = pbuf[sl]
            av = abuf[sl]
            sv = jnp.bitwise_and(pv, jnp.int32(0x3FFF))
            dv = lax.shift_right_logical(pv, 14)
            qv = plsc.load_gather(qtab, [sv])
            plsc.addupdate_scatter(
                acc, [lax.shift_right_logical(dv, 7),
                      jnp.bitwise_and(dv, jnp.int32(127))], av * qv)

        plsc.subcore_barrier()

        @pl.when(w < 10)
        def _():
            pltpu.sync_copy(zeros_h.at[pl.ds(w * 8, 8), :],
                            xacc.at[1 - p, pl.ds(w * 8, 8), :])

        pltpu.sync_copy(acc, xacc.at[p].at[idxb], add=True)
        plsc.subcore_barrier()

        pltpu.sync_copy(xacc.at[p], xful)
        pltpu.async_copy(zeros_h, acc, zsem)

        def hz(i, _):
            hist[pl.ds(i * L, L)] = zeros16i
            return 0
        lax.fori_loop(0, 256 // L, hz, 0)

        startv = sbuf[pl.ds(0, L)]

        @plsc.parallel_loop(0, NVR, 1, unroll=4)
        def _(i):
            raw = xful[i // 8, pl.ds((i % 8) * L, L)]
            sval = jnp.where(i * L + lanes == startv, ALPHA, 0.0)
            xv = (1.0 - ALPHA) * raw - sval
            xful[i // 8, pl.ds((i % 8) * L, L)] = xv
            b = 255 - lax.shift_right_logical(_mbits(xv), 23)
            plsc.addupdate_scatter(hist, [b], ones16i)

        b1, r1 = _hist_rank_select(hist, jnp.full((L,), K, jnp.int32))

        def radix_rest():
            @plsc.parallel_loop(0, NVR, 1, unroll=4,
                                carry=jnp.zeros((L,), jnp.int32))
            def cnt_v(i, off):
                m = _mbits(xful[i // 8, pl.ds((i % 8) * L, L)])
                keep = lax.shift_right_logical(m, 23) == b1
                ki = keep.astype(jnp.int32)
                cs = plsc.cumsum(ki)
                plsc.store_scatter(cbuf, [off + cs - ki], m, mask=keep)
                return off + plsc.all_reduce_population_count(keep)

            nv = jnp.max((cnt_v + (L - 1)) // L)

            def rpass(pref, pref_shift, buck_shift, buck_mask, rank):
                def hz2(i, _):
                    hist[pl.ds(i * L, L)] = zeros16i
                    return 0
                lax.fori_loop(0, 256 // L, hz2, 0)

                def pb(i, _):
                    m = cbuf[pl.ds(i * L, L)]
                    keep = jnp.logical_and(
                        lanes < cnt_v - i * L,
                        lax.shift_right_logical(m, pref_shift) == pref)
                    b = 255 - jnp.bitwise_and(
                        lax.shift_right_logical(m, buck_shift),
                        jnp.int32(buck_mask))
                    plsc.addupdate_scatter(hist, [b], ones16i, mask=keep)
                    return 0
                lax.fori_loop(0, nv, pb, 0)
                return _hist_rank_select(hist, rank)

            b2, r2 = rpass(b1, 23, 15, 0xFF, r1)
            p2 = b1 * 256 + b2
            b3, r3 = rpass(p2, 15, 7, 0xFF, r2)
            p3 = p2 * 256 + b3
            b4, _ = rpass(p3, 7, 0, 0x7F, r3)
            return p3 * 128 + b4

        t = lax.cond(jnp.max(b1) > 0, radix_rest,
                     lambda: jnp.ones((L,), jnp.int32))

        @plsc.parallel_loop(0, NVR, 1, unroll=4)
        def _(i):
            x = xful[i // 8, pl.ds((i % 8) * L, L)]
            val = jnp.sign(x) * jnp.maximum(jnp.abs(x) - THR, 0.0)
            qtab[pl.ds(i * L, L)] = jnp.where(_mbits(x) >= t, val, 0.0)

        return carry

    lax.fori_loop(0, NUM_STEPS, step, 0)
    pltpu.make_async_copy(zeros_h, acc, zsem).wait()

    @pl.when(c == 0)
    def _():
        pltpu.sync_copy(qtab.at[pl.ds(w * (NP // NS), NP // NS)],
                        qout_h.at[pl.ds(w * (NP // NS), NP // NS)])


@jax.jit
def _run(packed, data, svec, startv, zvec):
    mesh = plsc.VectorSubcoreMesh(core_axis_name="c", subcore_axis_name="s",
                                  num_cores=NC, num_subcores=NS)
    f = pl.kernel(
        _sc_body,
        out_type=[
            jax.ShapeDtypeStruct((NP,), jnp.float32),
        ],
        mesh=mesh,
        compiler_params=pltpu.CompilerParams(needs_layout_passes=False),
        scratch_types=[
            pltpu.VMEM((EPT,), jnp.int32),
            pltpu.VMEM((EPT,), jnp.float32),
            pltpu.VMEM((NP,), jnp.float32),
            pltpu.VMEM((NR, 128), jnp.float32),
            pltpu.VMEM((NR, 128), jnp.float32),
            pltpu.VMEM((NP + L,), jnp.int32),
            pltpu.VMEM((NR,), jnp.int32),
            pltpu.VMEM((L,), jnp.int32),
            pltpu.VMEM((256,), jnp.int32),
            pltpu.VMEM_SHARED((2, NR, 128), jnp.float32),
            pltpu.SemaphoreType.DMA,
        ],
    )
    q, = f(packed, data, svec, startv, zvec)
    return q[:N]


def kernel(adj_data, adj_indices, start_node_id):
    src = adj_indices[:, 0].astype(jnp.int32)
    dst = adj_indices[:, 1].astype(jnp.int32)
    packed = jnp.bitwise_or(src, jnp.left_shift(dst, 14))
    svec = jnp.zeros((NP,), jnp.float32).at[start_node_id].set(1.0)
    startv = jnp.full((L,), start_node_id, jnp.int32)
    zvec = jnp.zeros((NR, 128), jnp.float32)
    return _run(packed, adj_data.astype(jnp.float32), svec, startv, zvec)
